# baseline (speedup 1.0000x reference)
import functools

import jax
import jax.numpy as jnp
from jax import lax
from jax.experimental import pallas as pl
from jax.experimental.pallas import tpu as pltpu
from jax.experimental.pallas import tpu_sc as plsc

_B, _T, _V = 64, 32, 32768
_N = _B * _T


def _sc_gather(flat_inp, tgt_flat):
    info = plsc.get_sparse_core_info()
    nc, ns = info.num_cores, info.num_subcores
    nw = nc * ns
    per_w = _N // nw
    mesh = plsc.VectorSubcoreMesh(core_axis_name="c", subcore_axis_name="s")

    @functools.partial(
        pl.kernel,
        mesh=mesh,
        out_type=jax.ShapeDtypeStruct((_N,), jnp.float32),
        scratch_types=[
            pltpu.VMEM((per_w,), jnp.int32),
            pltpu.VMEM((per_w,), jnp.int32),
            pltpu.VMEM((per_w,), jnp.float32),
            pltpu.SemaphoreType.DMA,
        ],
    )
    def body(flat_hbm, tgt_hbm, out_hbm, tgt_v, idx_v, vals_v, sem):
        wid = lax.axis_index("s") * nc + lax.axis_index("c")
        base = wid * per_w
        pltpu.sync_copy(tgt_hbm.at[pl.ds(base, per_w)], tgt_v)
        for j in range(per_w // 16):
            t = tgt_v[pl.ds(j * 16, 16)]
            row = base + j * 16 + lax.iota(jnp.int32, 16)
            idx_v[pl.ds(j * 16, 16)] = row * _V + t
        pltpu.async_copy(flat_hbm.at[idx_v], vals_v, sem).wait()
        pltpu.sync_copy(vals_v, out_hbm.at[pl.ds(base, per_w)])

    return body(flat_inp, tgt_flat)


def _tc_loss(vals_ref, mask_ref, out_ref):
    v = vals_ref[...]
    m = mask_ref[...]
    ce = -jnp.log(v)
    out_ref[0, 0] = jnp.sum(ce * m) / jnp.sum(m)


def kernel(input, target, mask):
    flat = input.reshape(-1)
    tgt = target.reshape(-1).astype(jnp.int32)
    vals = _sc_gather(flat, tgt)
    mask_f = mask.reshape(16, 128).astype(jnp.float32)
    out = pl.pallas_call(
        _tc_loss,
        out_shape=jax.ShapeDtypeStruct((1, 1), jnp.float32),
        out_specs=pl.BlockSpec(memory_space=pltpu.SMEM),
    )(vals.reshape(16, 128), mask_f)
    return out[0, 0]


# speedup vs baseline: 7.9167x; 7.9167x over previous
import functools

import jax
import jax.numpy as jnp
from jax import lax
from jax.experimental import pallas as pl
from jax.experimental.pallas import tpu as pltpu
from jax.experimental.pallas import tpu_sc as plsc

_B, _T, _V = 64, 32, 32768
_N = _B * _T


def _sc_gather(inp, tgt_flat):
    info = plsc.get_sparse_core_info()
    nc, ns = info.num_cores, info.num_subcores
    nw = nc * ns
    per_w = _N // nw
    ngrp = per_w // 16
    mesh = plsc.VectorSubcoreMesh(core_axis_name="c", subcore_axis_name="s")

    @functools.partial(
        pl.kernel,
        mesh=mesh,
        out_type=jax.ShapeDtypeStruct((_N,), jnp.float32),
        compiler_params=pltpu.CompilerParams(needs_layout_passes=False),
        scratch_types=[
            pltpu.VMEM((per_w,), jnp.int32),
            pltpu.VMEM((per_w * 8, 128), jnp.float32),
            pltpu.VMEM((per_w,), jnp.float32),
            pltpu.SemaphoreType.DMA,
        ],
    )
    def body(inp_hbm, tgt_hbm, out_hbm, tgt_v, buf_v, vals_v, sem):
        wid = lax.axis_index("s") * nc + lax.axis_index("c")
        base = wid * per_w
        pltpu.sync_copy(tgt_hbm.at[pl.ds(base, per_w)], tgt_v)
        lane = lax.iota(jnp.int32, 16)
        copies = []
        for j in range(ngrp):
            t16 = tgt_v[pl.ds(j * 16, 16)]
            for i in range(16):
                e = j * 16 + i
                b_ix = (base + e) // _T
                t0 = (e % _T) // 8 * 8
                v = t16[i]
                v0 = pl.multiple_of(jnp.bitwise_and(v, -128), 128)
                copies.append(
                    pltpu.async_copy(
                        inp_hbm.at[b_ix, pl.ds(t0, 8), pl.ds(v0, 128)],
                        buf_v.at[pl.ds(e * 8, 8), :],
                        sem,
                    )
                )
        for c in copies:
            c.wait()
        row_in_tile = jnp.bitwise_and(lane, 7)
        for j in range(ngrp):
            t16 = tgt_v[pl.ds(j * 16, 16)]
            rows = (j * 16 + lane) * 8 + row_in_tile
            col = jnp.bitwise_and(t16, 127)
            vals_v[pl.ds(j * 16, 16)] = plsc.load_gather(buf_v, [rows, col])
        pltpu.sync_copy(vals_v, out_hbm.at[pl.ds(base, per_w)])

    return body(inp, tgt_flat)


def _tc_loss(vals_ref, mask_ref, out_ref):
    v = vals_ref[...]
    m = mask_ref[...]
    ce = -jnp.log(v)
    out_ref[0, 0] = jnp.sum(ce * m) / jnp.sum(m)


def kernel(input, target, mask):
    tgt = target.reshape(-1).astype(jnp.int32)
    vals = _sc_gather(input, tgt)
    mask_f = mask.reshape(16, 128).astype(jnp.float32)
    out = pl.pallas_call(
        _tc_loss,
        out_shape=jax.ShapeDtypeStruct((1, 1), jnp.float32),
        out_specs=pl.BlockSpec(memory_space=pltpu.SMEM),
    )(vals.reshape(16, 128), mask_f)
    return out[0, 0]


# speedup vs baseline: 9.2687x; 1.1708x over previous
import functools

import jax
import jax.numpy as jnp
from jax import lax
from jax.experimental import pallas as pl
from jax.experimental.pallas import tpu as pltpu
from jax.experimental.pallas import tpu_sc as plsc

_B, _T, _V = 64, 32, 32768
_N = _B * _T


def _sc_gather(rows_view, tgt_flat):
    info = plsc.get_sparse_core_info()
    nc, ns = info.num_cores, info.num_subcores
    nw = nc * ns
    per_w = _N // nw
    ngrp = per_w // 16
    mesh = plsc.VectorSubcoreMesh(core_axis_name="c", subcore_axis_name="s")

    @functools.partial(
        pl.kernel,
        mesh=mesh,
        out_type=jax.ShapeDtypeStruct((_N,), jnp.float32),
        compiler_params=pltpu.CompilerParams(needs_layout_passes=False),
        scratch_types=[
            pltpu.VMEM((per_w,), jnp.int32),
            pltpu.VMEM((per_w,), jnp.int32),
            pltpu.VMEM((per_w, 128), jnp.float32),
            pltpu.VMEM((per_w,), jnp.float32),
            pltpu.SemaphoreType.DMA,
        ],
    )
    def body(rows_hbm, tgt_hbm, out_hbm, tgt_v, idx_v, buf_v, vals_v, sem):
        wid = lax.axis_index("s") * nc + lax.axis_index("c")
        base = wid * per_w
        pltpu.sync_copy(tgt_hbm.at[pl.ds(base, per_w)], tgt_v)
        lane = lax.iota(jnp.int32, 16)
        for j in range(ngrp):
            tv = tgt_v[pl.ds(j * 16, 16)]
            e16 = base + j * 16 + lane
            b16 = jnp.right_shift(e16, 5)
            tpos = jnp.bitwise_and(e16, 31)
            const = (
                b16 * 8192
                + jnp.right_shift(tpos, 3) * 2048
                + jnp.bitwise_and(tpos, 7)
            )
            idx_v[pl.ds(j * 16, 16)] = const + jnp.right_shift(tv, 7) * 8
        pltpu.async_copy(rows_hbm.at[idx_v], buf_v, sem).wait()
        for j in range(ngrp):
            tv = tgt_v[pl.ds(j * 16, 16)]
            col = jnp.bitwise_and(tv, 127)
            vals_v[pl.ds(j * 16, 16)] = plsc.load_gather(
                buf_v, [j * 16 + lane, col]
            )
        pltpu.sync_copy(vals_v, out_hbm.at[pl.ds(base, per_w)])

    return body(rows_view, tgt_flat)


def _tc_loss(vals_ref, mask_ref, out_ref):
    v = vals_ref[...]
    m = mask_ref[...]
    ce = -jnp.log(v)
    out_ref[0, 0] = jnp.sum(ce * m) / jnp.sum(m)


def kernel(input, target, mask):
    rows_view = (
        input.reshape(_B, _T // 8, 8, _V // 128, 128)
        .transpose(0, 1, 3, 2, 4)
        .reshape(_B * (_T // 8) * (_V // 128) * 8, 128)
    )
    tgt = target.reshape(-1).astype(jnp.int32)
    vals = _sc_gather(rows_view, tgt)
    mask_f = mask.reshape(16, 128).astype(jnp.float32)
    out = pl.pallas_call(
        _tc_loss,
        out_shape=jax.ShapeDtypeStruct((1, 1), jnp.float32),
        out_specs=pl.BlockSpec(memory_space=pltpu.SMEM),
    )(vals.reshape(16, 128), mask_f)
    return out[0, 0]
